# baseline (speedup 1.0000x reference)
import functools

import jax
import jax.numpy as jnp
from jax import lax
from jax.experimental import pallas as pl
from jax.experimental.pallas import tpu as pltpu, tpu_sc as plsc

B = 8
N = 20000
C = 80
K = 2000
PAD_ROWS = 8
NP_ = N + PAD_ROWS

SCORE_THRESHOLD = 0.05
KEY_BASE = 0x3D4CCCCD
DIGIT_BITS = 13
DIGIT_MASK = (1 << DIGIT_BITS) - 1
NUM_BINS = 1 << DIGIT_BITS

BN = 2000


def _tc_body(cls_ref, nk_ref, pay_ref):
    x = cls_ref[0]
    m = jnp.max(x, axis=1, keepdims=True)
    key = lax.bitcast_convert_type(m, jnp.int32)
    valid = m > jnp.float32(SCORE_THRESHOLD)
    nk = jnp.where(valid, key - KEY_BASE, 0)
    il = lax.broadcasted_iota(jnp.int32, (BN, C), 1)
    lbl = jnp.min(
        jnp.where(x == m, il, jnp.int32(C * 2)), axis=1, keepdims=True
    )
    bi = pl.program_id(1) * BN + lax.broadcasted_iota(jnp.int32, (BN, 1), 0)
    nk_ref[0, 0] = nk
    pay_ref[0, 0] = (bi << 7) | lbl


def _tc_stage(classification):
    grid = (B, N // BN)
    return pl.pallas_call(
        _tc_body,
        grid=grid,
        in_specs=[
            pl.BlockSpec((1, BN, C), lambda b, j: (b, j, 0)),
        ],
        out_specs=[
            pl.BlockSpec((1, 1, BN, 1), lambda b, j: (b, j, 0, 0)),
            pl.BlockSpec((1, 1, BN, 1), lambda b, j: (b, j, 0, 0)),
        ],
        out_shape=[
            jax.ShapeDtypeStruct((B, N // BN, BN, 1), jnp.int32),
            jax.ShapeDtypeStruct((B, N // BN, BN, 1), jnp.int32),
        ],
    )(classification)


def _counting_pass(src_nk, src_pay, dst_nk, dst_pay, hist, shift, badj):
    def zero_body(i, _):
        hist[pl.ds(i * 16, 16)] = jnp.zeros((16,), jnp.int32)
        return 0

    lax.fori_loop(0, NUM_BINS // 16, zero_body, 0)

    def hist_body(i, _):
        v = src_nk[pl.ds(i * 16, 16)]
        d = DIGIT_MASK - ((v >> shift) & DIGIT_MASK)
        cnt, last = plsc.scan_count(d)
        total = cnt - badj + 1
        plsc.addupdate_scatter(hist, [d], total, mask=last)
        return 0

    lax.fori_loop(0, N // 16, hist_body, 0)

    def scan_body(i, carry):
        v = hist[pl.ds(i * 16, 16)]
        cs = plsc.cumsum(v)
        hist[pl.ds(i * 16, 16)] = cs - v + carry
        return carry + jnp.max(cs)

    lax.fori_loop(0, NUM_BINS // 16, scan_body, jnp.int32(0))

    def perm_body(i, _):
        v = src_nk[pl.ds(i * 16, 16)]
        p = src_pay[pl.ds(i * 16, 16)]
        d = DIGIT_MASK - ((v >> shift) & DIGIT_MASK)
        cnt, last = plsc.scan_count(d)
        rank = cnt - badj
        cur = plsc.load_gather(hist, [d])
        off = cur + rank
        plsc.store_scatter(dst_nk, [off], v)
        plsc.store_scatter(dst_pay, [off], p)
        plsc.store_scatter(hist, [d], off + 1, mask=last)
        return 0

    lax.fori_loop(0, N // 16, perm_body, 0)


def _sc_body(
    nk_hbm, pay_hbm, boxes_hbm,
    boxes_out, scores_out, labels_out,
    nk_a, pay_a, nk_b, pay_b, hist, gidx, sco_v, lbl_v, box_v, sem,
):
    c = lax.axis_index("c")
    s = lax.axis_index("s")
    wid = s * 2 + c

    @pl.when(wid < B)
    def _():
        b = wid
        pltpu.sync_copy(nk_hbm.at[b], nk_a)
        pltpu.sync_copy(pay_hbm.at[b], pay_a)

        cnt0, _ = plsc.scan_count(jnp.zeros((16,), jnp.int32))
        badj = jnp.min(cnt0)

        _counting_pass(nk_a, pay_a, nk_b, pay_b, hist, 0, badj)
        _counting_pass(nk_b, pay_b, nk_a, pay_a, hist, DIGIT_BITS, badj)

        def out_body(i, _):
            v = nk_a[pl.ds(i * 16, 16)]
            p = pay_a[pl.ds(i * 16, 16)]
            valid = v > 0
            sco_v[pl.ds(i * 16, 16)] = jnp.where(
                valid, plsc.bitcast(v + KEY_BASE, jnp.float32),
                jnp.float32(-1.0))
            lbl_v[pl.ds(i * 16, 16)] = jnp.where(valid, p & 127, -1)
            sent = N + (lax.iota(jnp.int32, 16) & (PAD_ROWS - 1))
            row = jnp.where(valid, p >> 7, sent)
            k6 = (i * 16 + lax.iota(jnp.int32, 16)) * 6
            r6 = row * 6
            for cc in range(6):
                plsc.store_scatter(gidx, [k6 + cc], r6 + cc)
            return 0

        lax.fori_loop(0, K // 16, out_body, 0)

        pltpu.sync_copy(sco_v, scores_out.at[b])
        pltpu.sync_copy(lbl_v, labels_out.at[b])

        pltpu.async_copy(boxes_hbm.at[b].at[gidx], box_v, sem).wait()
        pltpu.sync_copy(box_v, boxes_out.at[b])


def _sc_stage(nk, pay, boxes_pad):
    mesh = plsc.VectorSubcoreMesh(core_axis_name="c", subcore_axis_name="s")
    kern = pl.kernel(
        _sc_body,
        out_type=(
            jax.ShapeDtypeStruct((B, K * 6), jnp.float32),
            jax.ShapeDtypeStruct((B, K), jnp.float32),
            jax.ShapeDtypeStruct((B, K), jnp.int32),
        ),
        mesh=mesh,
        compiler_params=pltpu.CompilerParams(
            needs_layout_passes=False, use_tc_tiling_on_sc=False),
        scratch_types=[
            pltpu.VMEM((N,), jnp.int32),
            pltpu.VMEM((N,), jnp.int32),
            pltpu.VMEM((N,), jnp.int32),
            pltpu.VMEM((N,), jnp.int32),
            pltpu.VMEM((NUM_BINS,), jnp.int32),
            pltpu.VMEM((K * 6,), jnp.int32),
            pltpu.VMEM((K,), jnp.float32),
            pltpu.VMEM((K,), jnp.int32),
            pltpu.VMEM((K * 6,), jnp.float32),
            pltpu.SemaphoreType.DMA,
        ],
    )
    return kern(nk, pay, boxes_pad)


def kernel(boxes, classification):
    nk4, pay4 = _tc_stage(classification)
    nk = nk4.reshape(B, N)
    pay = pay4.reshape(B, N)
    boxes_pad = jnp.pad(
        boxes, ((0, 0), (0, PAD_ROWS), (0, 0)), constant_values=-1.0
    ).reshape(B, NP_ * 6)
    boxes_flat, scores_out, labels_out = _sc_stage(nk, pay, boxes_pad)
    return boxes_flat.reshape(B, K, 6), scores_out, labels_out


# speedup vs baseline: 1.2135x; 1.2135x over previous
import functools

import jax
import jax.numpy as jnp
from jax import lax
from jax.experimental import pallas as pl
from jax.experimental.pallas import tpu as pltpu, tpu_sc as plsc

B = 8
N = 20000
C = 80
K = 2000
PAD_ROWS = 8
NP_ = N + PAD_ROWS

SCORE_THRESHOLD = 0.05
KEY_BASE = 0x3D4CCCCD
DIGIT_BITS = 13
DIGIT_MASK = (1 << DIGIT_BITS) - 1
NUM_BINS = 1 << DIGIT_BITS

BN = 2000


def _tc_body(cls_ref, nk_ref, pay_ref):
    x = cls_ref[0]
    xt = x.T
    m = jnp.max(xt, axis=0, keepdims=True)
    key = lax.bitcast_convert_type(m, jnp.int32)
    valid = m > jnp.float32(SCORE_THRESHOLD)
    nk = jnp.where(valid, key - KEY_BASE, 0)
    ir = lax.broadcasted_iota(jnp.int32, (C, BN), 0)
    lbl = jnp.min(
        jnp.where(xt == m, ir, jnp.int32(C * 2)), axis=0, keepdims=True
    )
    bi = pl.program_id(1) * BN + lax.broadcasted_iota(jnp.int32, (1, BN), 1)
    nk_ref[0] = nk
    pay_ref[0] = (bi << 7) | lbl


def _tc_stage(classification):
    grid = (B, N // BN)
    return pl.pallas_call(
        _tc_body,
        grid=grid,
        in_specs=[
            pl.BlockSpec((1, BN, C), lambda b, j: (b, j, 0)),
        ],
        out_specs=[
            pl.BlockSpec((1, 1, BN), lambda b, j: (b * (N // BN) + j, 0, 0)),
            pl.BlockSpec((1, 1, BN), lambda b, j: (b * (N // BN) + j, 0, 0)),
        ],
        out_shape=[
            jax.ShapeDtypeStruct((B * (N // BN), 1, BN), jnp.int32),
            jax.ShapeDtypeStruct((B * (N // BN), 1, BN), jnp.int32),
        ],
    )(classification)


def _counting_pass(src_nk, src_pay, dst_nk, dst_pay, hist, shift, badj):
    def zero_body(i, _):
        hist[pl.ds(i * 16, 16)] = jnp.zeros((16,), jnp.int32)
        return 0

    lax.fori_loop(0, NUM_BINS // 16, zero_body, 0)

    def hist_body(i, _):
        v = src_nk[pl.ds(i * 16, 16)]
        d = DIGIT_MASK - ((v >> shift) & DIGIT_MASK)
        cnt, last = plsc.scan_count(d)
        total = cnt - badj + 1
        plsc.addupdate_scatter(hist, [d], total, mask=last)
        return 0

    lax.fori_loop(0, N // 16, hist_body, 0)

    def scan_body(i, carry):
        v = hist[pl.ds(i * 16, 16)]
        cs = plsc.cumsum(v)
        hist[pl.ds(i * 16, 16)] = cs - v + carry
        return carry + jnp.max(cs)

    lax.fori_loop(0, NUM_BINS // 16, scan_body, jnp.int32(0))

    def perm_body(i, _):
        v = src_nk[pl.ds(i * 16, 16)]
        p = src_pay[pl.ds(i * 16, 16)]
        d = DIGIT_MASK - ((v >> shift) & DIGIT_MASK)
        cnt, last = plsc.scan_count(d)
        rank = cnt - badj
        cur = plsc.load_gather(hist, [d])
        off = cur + rank
        plsc.store_scatter(dst_nk, [off], v)
        plsc.store_scatter(dst_pay, [off], p)
        plsc.store_scatter(hist, [d], off + 1, mask=last)
        return 0

    lax.fori_loop(0, N // 16, perm_body, 0)


def _sc_body(
    nk_hbm, pay_hbm, boxes_hbm,
    boxes_out, scores_out, labels_out,
    nk_a, pay_a, nk_b, pay_b, hist, gidx, sco_v, lbl_v, box_v, sem,
):
    c = lax.axis_index("c")
    s = lax.axis_index("s")
    wid = s * 2 + c

    @pl.when(wid < B)
    def _():
        b = wid
        pltpu.sync_copy(nk_hbm.at[b], nk_a)
        pltpu.sync_copy(pay_hbm.at[b], pay_a)

        cnt0, _ = plsc.scan_count(jnp.zeros((16,), jnp.int32))
        badj = jnp.min(cnt0)

        _counting_pass(nk_a, pay_a, nk_b, pay_b, hist, 0, badj)
        _counting_pass(nk_b, pay_b, nk_a, pay_a, hist, DIGIT_BITS, badj)

        def out_body(i, _):
            v = nk_a[pl.ds(i * 16, 16)]
            p = pay_a[pl.ds(i * 16, 16)]
            valid = v > 0
            sco_v[pl.ds(i * 16, 16)] = jnp.where(
                valid, plsc.bitcast(v + KEY_BASE, jnp.float32),
                jnp.float32(-1.0))
            lbl_v[pl.ds(i * 16, 16)] = jnp.where(valid, p & 127, -1)
            sent = N + (lax.iota(jnp.int32, 16) & (PAD_ROWS - 1))
            row = jnp.where(valid, p >> 7, sent)
            k6 = (i * 16 + lax.iota(jnp.int32, 16)) * 6
            r6 = row * 6
            for cc in range(6):
                plsc.store_scatter(gidx, [k6 + cc], r6 + cc)
            return 0

        lax.fori_loop(0, K // 16, out_body, 0)

        pltpu.sync_copy(sco_v, scores_out.at[b])
        pltpu.sync_copy(lbl_v, labels_out.at[b])

        pltpu.async_copy(boxes_hbm.at[b].at[gidx], box_v, sem).wait()
        pltpu.sync_copy(box_v, boxes_out.at[b])


def _sc_stage(nk, pay, boxes_pad):
    mesh = plsc.VectorSubcoreMesh(core_axis_name="c", subcore_axis_name="s")
    kern = pl.kernel(
        _sc_body,
        out_type=(
            jax.ShapeDtypeStruct((B, K * 6), jnp.float32),
            jax.ShapeDtypeStruct((B, K), jnp.float32),
            jax.ShapeDtypeStruct((B, K), jnp.int32),
        ),
        mesh=mesh,
        compiler_params=pltpu.CompilerParams(
            needs_layout_passes=False, use_tc_tiling_on_sc=False),
        scratch_types=[
            pltpu.VMEM((N,), jnp.int32),
            pltpu.VMEM((N,), jnp.int32),
            pltpu.VMEM((N,), jnp.int32),
            pltpu.VMEM((N,), jnp.int32),
            pltpu.VMEM((NUM_BINS,), jnp.int32),
            pltpu.VMEM((K * 6,), jnp.int32),
            pltpu.VMEM((K,), jnp.float32),
            pltpu.VMEM((K,), jnp.int32),
            pltpu.VMEM((K * 6,), jnp.float32),
            pltpu.SemaphoreType.DMA,
        ],
    )
    return kern(nk, pay, boxes_pad)


def kernel(boxes, classification):
    nk4, pay4 = _tc_stage(classification)
    nk = nk4.reshape(B, N)
    pay = pay4.reshape(B, N)
    boxes_pad = jnp.pad(
        boxes, ((0, 0), (0, PAD_ROWS), (0, 0)), constant_values=-1.0
    ).reshape(B, NP_ * 6)
    boxes_flat, scores_out, labels_out = _sc_stage(nk, pay, boxes_pad)
    return boxes_flat.reshape(B, K, 6), scores_out, labels_out


# speedup vs baseline: 1.3314x; 1.0971x over previous
import functools

import jax
import jax.numpy as jnp
from jax import lax
from jax.experimental import pallas as pl
from jax.experimental.pallas import tpu as pltpu, tpu_sc as plsc

B = 8
N = 20000
C = 80
K = 2000
PAD_ROWS = 8
NP_ = N + PAD_ROWS

SCORE_THRESHOLD = 0.05
KEY_BASE = 0x3D4CCCCD
DIGIT_BITS = 13
DIGIT_MASK = (1 << DIGIT_BITS) - 1
NUM_BINS = 1 << DIGIT_BITS

BN = 2000


def _tc_body(cls_ref, nk_ref, pay_ref):
    x = cls_ref[0]
    xt = x.T
    m = jnp.max(xt, axis=0, keepdims=True)
    key = lax.bitcast_convert_type(m, jnp.int32)
    valid = m > jnp.float32(SCORE_THRESHOLD)
    nk = jnp.where(valid, key - KEY_BASE, 0)
    ir = lax.broadcasted_iota(jnp.int32, (C, BN), 0)
    lbl = jnp.min(
        jnp.where(xt == m, ir, jnp.int32(C * 2)), axis=0, keepdims=True
    )
    bi = pl.program_id(1) * BN + lax.broadcasted_iota(jnp.int32, (1, BN), 1)
    nk_ref[0] = nk
    pay_ref[0] = (bi << 7) | lbl


def _tc_stage(classification):
    grid = (B, N // BN)
    return pl.pallas_call(
        _tc_body,
        grid=grid,
        in_specs=[
            pl.BlockSpec((1, BN, C), lambda b, j: (b, j, 0)),
        ],
        out_specs=[
            pl.BlockSpec((1, 1, BN), lambda b, j: (b * (N // BN) + j, 0, 0)),
            pl.BlockSpec((1, 1, BN), lambda b, j: (b * (N // BN) + j, 0, 0)),
        ],
        out_shape=[
            jax.ShapeDtypeStruct((B * (N // BN), 1, BN), jnp.int32),
            jax.ShapeDtypeStruct((B * (N // BN), 1, BN), jnp.int32),
        ],
    )(classification)


LO_BITS = 9
LO_MASK = (1 << LO_BITS) - 1
LO_BINS = 1 << LO_BITS


def _counting_pass(src_nk, src_pay, dst_nk, dst_pay, hist, shift, badj, nvec):
    def zero_body(i, _):
        hist[pl.ds(i * 16, 16)] = jnp.zeros((16,), jnp.int32)
        return 0

    lax.fori_loop(0, LO_BINS // 16, zero_body, 0)

    def hist_body(i, _):
        v = src_nk[pl.ds(i * 16, 16)]
        d = LO_MASK - ((v >> shift) & LO_MASK)
        cnt, last = plsc.scan_count(d)
        total = cnt - badj + 1
        plsc.addupdate_scatter(hist, [d], total, mask=last)
        return 0

    lax.fori_loop(0, nvec, hist_body, 0)

    def scan_body(i, carry):
        v = hist[pl.ds(i * 16, 16)]
        cs = plsc.cumsum(v)
        hist[pl.ds(i * 16, 16)] = cs - v + carry
        return carry + jnp.max(cs)

    lax.fori_loop(0, LO_BINS // 16, scan_body, jnp.int32(0))

    def perm_body(i, _):
        v = src_nk[pl.ds(i * 16, 16)]
        p = src_pay[pl.ds(i * 16, 16)]
        d = LO_MASK - ((v >> shift) & LO_MASK)
        cnt, last = plsc.scan_count(d)
        rank = cnt - badj
        cur = plsc.load_gather(hist, [d])
        off = cur + rank
        plsc.store_scatter(dst_nk, [off], v)
        plsc.store_scatter(dst_pay, [off], p)
        plsc.store_scatter(hist, [d], off + 1, mask=last)
        return 0

    lax.fori_loop(0, nvec, perm_body, 0)


def _sc_body(
    nk_hbm, pay_hbm, boxes_hbm,
    boxes_out, scores_out, labels_out,
    nk_a, pay_a, cand_nk, cand_pay, hist, gidx, sco_v, lbl_v, box_v, sem,
):
    c = lax.axis_index("c")
    s = lax.axis_index("s")
    wid = s * 2 + c

    @pl.when(wid < B)
    def _():
        b = wid
        pltpu.sync_copy(nk_hbm.at[b], nk_a)
        pltpu.sync_copy(pay_hbm.at[b], pay_a)

        cnt0, _ = plsc.scan_count(jnp.zeros((16,), jnp.int32))
        badj = jnp.min(cnt0)

        def zero_body(i, _):
            hist[pl.ds(i * 16, 16)] = jnp.zeros((16,), jnp.int32)
            return 0

        lax.fori_loop(0, NUM_BINS // 16, zero_body, 0)

        def hist_body(i, _):
            v = nk_a[pl.ds(i * 16, 16)]
            d = DIGIT_MASK - ((v >> DIGIT_BITS) & DIGIT_MASK)
            cnt, last = plsc.scan_count(d)
            plsc.addupdate_scatter(hist, [d], cnt - badj + 1, mask=last)
            return 0

        lax.fori_loop(0, N // 16, hist_body, 0)

        def cut_body(i, carry):
            cut, run = carry
            v = hist[pl.ds(i * 16, 16)]
            cs = plsc.cumsum(v) + run
            pc = jnp.max(plsc.all_reduce_population_count(cs >= K))
            cutl = i * 16 + (16 - pc)
            cut = jnp.minimum(
                cut, jnp.where(pc > 0, cutl, jnp.int32(NUM_BINS)))
            return cut, jnp.max(cs)

        cut, _ = lax.fori_loop(
            0, NUM_BINS // 16, cut_body,
            (jnp.int32(NUM_BINS), jnp.int32(0)))

        def compact_body(i, off):
            v = nk_a[pl.ds(i * 16, 16)]
            p = pay_a[pl.ds(i * 16, 16)]
            d = DIGIT_MASK - ((v >> DIGIT_BITS) & DIGIT_MASK)
            m = d <= cut
            plsc.store_compressed(cand_nk.at[pl.ds(off, 16)], v, mask=m)
            plsc.store_compressed(cand_pay.at[pl.ds(off, 16)], p, mask=m)
            return off + jnp.max(plsc.all_reduce_population_count(m))

        mc = lax.fori_loop(0, N // 16, compact_body, jnp.int32(0))
        cand_nk[pl.ds(mc, 16)] = jnp.zeros((16,), jnp.int32)
        cand_pay[pl.ds(mc, 16)] = jnp.zeros((16,), jnp.int32)
        nvec = (mc + 15) >> 4

        _counting_pass(cand_nk, cand_pay, nk_a, pay_a, hist, 0, badj, nvec)
        _counting_pass(nk_a, pay_a, cand_nk, cand_pay, hist, LO_BITS, badj,
                       nvec)
        _counting_pass(cand_nk, cand_pay, nk_a, pay_a, hist, 2 * LO_BITS,
                       badj, nvec)

        def out_body(i, _):
            v = nk_a[pl.ds(i * 16, 16)]
            p = pay_a[pl.ds(i * 16, 16)]
            valid = v > 0
            sco_v[pl.ds(i * 16, 16)] = jnp.where(
                valid, plsc.bitcast(v + KEY_BASE, jnp.float32),
                jnp.float32(-1.0))
            lbl_v[pl.ds(i * 16, 16)] = jnp.where(valid, p & 127, -1)
            sent = N + (lax.iota(jnp.int32, 16) & (PAD_ROWS - 1))
            row = jnp.where(valid, p >> 7, sent)
            k6 = (i * 16 + lax.iota(jnp.int32, 16)) * 6
            r6 = row * 6
            for cc in range(6):
                plsc.store_scatter(gidx, [k6 + cc], r6 + cc)
            return 0

        lax.fori_loop(0, K // 16, out_body, 0)

        pltpu.sync_copy(sco_v, scores_out.at[b])
        pltpu.sync_copy(lbl_v, labels_out.at[b])

        pltpu.async_copy(boxes_hbm.at[b].at[gidx], box_v, sem).wait()
        pltpu.sync_copy(box_v, boxes_out.at[b])


def _sc_stage(nk, pay, boxes_pad):
    mesh = plsc.VectorSubcoreMesh(core_axis_name="c", subcore_axis_name="s")
    kern = pl.kernel(
        _sc_body,
        out_type=(
            jax.ShapeDtypeStruct((B, K * 6), jnp.float32),
            jax.ShapeDtypeStruct((B, K), jnp.float32),
            jax.ShapeDtypeStruct((B, K), jnp.int32),
        ),
        mesh=mesh,
        compiler_params=pltpu.CompilerParams(
            needs_layout_passes=False, use_tc_tiling_on_sc=False),
        scratch_types=[
            pltpu.VMEM((N,), jnp.int32),
            pltpu.VMEM((N,), jnp.int32),
            pltpu.VMEM((N + 16,), jnp.int32),
            pltpu.VMEM((N + 16,), jnp.int32),
            pltpu.VMEM((NUM_BINS,), jnp.int32),
            pltpu.VMEM((K * 6,), jnp.int32),
            pltpu.VMEM((K,), jnp.float32),
            pltpu.VMEM((K,), jnp.int32),
            pltpu.VMEM((K * 6,), jnp.float32),
            pltpu.SemaphoreType.DMA,
        ],
    )
    return kern(nk, pay, boxes_pad)


def kernel(boxes, classification):
    nk4, pay4 = _tc_stage(classification)
    nk = nk4.reshape(B, N)
    pay = pay4.reshape(B, N)
    boxes_pad = jnp.pad(
        boxes, ((0, 0), (0, PAD_ROWS), (0, 0)), constant_values=-1.0
    ).reshape(B, NP_ * 6)
    boxes_flat, scores_out, labels_out = _sc_stage(nk, pay, boxes_pad)
    return boxes_flat.reshape(B, K, 6), scores_out, labels_out


# speedup vs baseline: 1.6601x; 1.2469x over previous
import functools

import jax
import jax.numpy as jnp
from jax import lax
from jax.experimental import pallas as pl
from jax.experimental.pallas import tpu as pltpu, tpu_sc as plsc

B = 8
N = 20000
C = 80
K = 2000
PAD_ROWS = 8
NP_ = N + PAD_ROWS

SCORE_THRESHOLD = 0.05
KEY_BASE = 0x3D4CCCCD
DIGIT_BITS = 13
DIGIT_MASK = (1 << DIGIT_BITS) - 1
NUM_BINS = 1 << DIGIT_BITS

BN = 2000
CH = 2048
NCHUNK = N // BN
N2 = NCHUNK * CH


def _tc_body(cls_ref, nk_ref, pay_ref):
    j = pl.program_id(0)
    bi = j * BN + lax.broadcasted_iota(jnp.int32, (1, BN), 1)
    rows_nk = []
    rows_pay = []
    for bb in range(B):
        x = cls_ref[bb]
        xt = x.T
        m = jnp.max(xt, axis=0, keepdims=True)
        key = lax.bitcast_convert_type(m, jnp.int32)
        valid = m > jnp.float32(SCORE_THRESHOLD)
        nk = jnp.where(valid, key - KEY_BASE, 0)
        ir = lax.broadcasted_iota(jnp.int32, (C, BN), 0)
        lbl = jnp.min(
            jnp.where(xt == m, ir, jnp.int32(C * 2)), axis=0, keepdims=True
        )
        rows_nk.append(nk)
        rows_pay.append((bi << 7) | lbl)
    zpad = jnp.zeros((B, CH - BN), jnp.int32)
    nk_ref[0] = jnp.concatenate([jnp.concatenate(rows_nk, 0), zpad], 1)
    pay_ref[0] = jnp.concatenate([jnp.concatenate(rows_pay, 0), zpad], 1)


def _tc_stage(classification):
    grid = (N // BN,)
    return pl.pallas_call(
        _tc_body,
        grid=grid,
        in_specs=[
            pl.BlockSpec((B, BN, C), lambda j: (0, j, 0)),
        ],
        out_specs=[
            pl.BlockSpec((1, B, CH), lambda j: (j, 0, 0)),
            pl.BlockSpec((1, B, CH), lambda j: (j, 0, 0)),
        ],
        out_shape=[
            jax.ShapeDtypeStruct((N // BN, B, CH), jnp.int32),
            jax.ShapeDtypeStruct((N // BN, B, CH), jnp.int32),
        ],
    )(classification)


LO_BITS = 9
LO_MASK = (1 << LO_BITS) - 1
LO_BINS = 1 << LO_BITS


def _counting_pass(src_nk, src_pay, dst_nk, dst_pay, hist, shift, badj, nvec):
    def zero_body(i, _):
        hist[pl.ds(i * 16, 16)] = jnp.zeros((16,), jnp.int32)
        return 0

    lax.fori_loop(0, LO_BINS // 16, zero_body, 0)

    def hist_body(i, _):
        v = src_nk[pl.ds(i * 16, 16)]
        d = LO_MASK - ((v >> shift) & LO_MASK)
        cnt, last = plsc.scan_count(d)
        total = cnt - badj + 1
        plsc.addupdate_scatter(hist, [d], total, mask=last)
        return 0

    lax.fori_loop(0, nvec, hist_body, 0)

    def scan_body(i, carry):
        v = hist[pl.ds(i * 16, 16)]
        cs = plsc.cumsum(v)
        hist[pl.ds(i * 16, 16)] = cs - v + carry
        return carry + jnp.max(cs)

    lax.fori_loop(0, LO_BINS // 16, scan_body, jnp.int32(0))

    def perm_body(i, _):
        v = src_nk[pl.ds(i * 16, 16)]
        p = src_pay[pl.ds(i * 16, 16)]
        d = LO_MASK - ((v >> shift) & LO_MASK)
        cnt, last = plsc.scan_count(d)
        rank = cnt - badj
        cur = plsc.load_gather(hist, [d])
        off = cur + rank
        plsc.store_scatter(dst_nk, [off], v)
        plsc.store_scatter(dst_pay, [off], p)
        plsc.store_scatter(hist, [d], off + 1, mask=last)
        return 0

    lax.fori_loop(0, nvec, perm_body, 0)


def _sc_body(
    nk_hbm, pay_hbm, boxes_hbm,
    boxes_out, scores_out, labels_out,
    nk_a, pay_a, cand_nk, cand_pay, hist, gidx, sco_v, lbl_v, box_v, sem,
):
    c = lax.axis_index("c")
    s = lax.axis_index("s")
    wid = s * 2 + c

    @pl.when(wid < B)
    def _():
        b = wid
        for j in range(NCHUNK):
            pltpu.sync_copy(nk_hbm.at[j, b], nk_a.at[pl.ds(j * CH, CH)])
            pltpu.sync_copy(pay_hbm.at[j, b], pay_a.at[pl.ds(j * CH, CH)])

        cnt0, _ = plsc.scan_count(jnp.zeros((16,), jnp.int32))
        badj = jnp.min(cnt0)

        def zero_body(i, _):
            hist[pl.ds(i * 16, 16)] = jnp.zeros((16,), jnp.int32)
            return 0

        lax.fori_loop(0, NUM_BINS // 16, zero_body, 0)

        def hist_body(i, _):
            v = nk_a[pl.ds(i * 16, 16)]
            d = DIGIT_MASK - ((v >> DIGIT_BITS) & DIGIT_MASK)
            cnt, last = plsc.scan_count(d)
            plsc.addupdate_scatter(hist, [d], cnt - badj + 1, mask=last)
            return 0

        lax.fori_loop(0, N2 // 16, hist_body, 0)

        def cut_body(i, carry):
            cut, run = carry
            v = hist[pl.ds(i * 16, 16)]
            cs = plsc.cumsum(v) + run
            pc = jnp.max(plsc.all_reduce_population_count(cs >= K))
            cutl = i * 16 + (16 - pc)
            cut = jnp.minimum(
                cut, jnp.where(pc > 0, cutl, jnp.int32(NUM_BINS)))
            return cut, jnp.max(cs)

        cut, _ = lax.fori_loop(
            0, NUM_BINS // 16, cut_body,
            (jnp.int32(NUM_BINS), jnp.int32(0)))

        def compact_body(i, off):
            v = nk_a[pl.ds(i * 16, 16)]
            p = pay_a[pl.ds(i * 16, 16)]
            d = DIGIT_MASK - ((v >> DIGIT_BITS) & DIGIT_MASK)
            m = d <= cut
            plsc.store_compressed(cand_nk.at[pl.ds(off, 16)], v, mask=m)
            plsc.store_compressed(cand_pay.at[pl.ds(off, 16)], p, mask=m)
            return off + jnp.max(plsc.all_reduce_population_count(m))

        mc = lax.fori_loop(0, N2 // 16, compact_body, jnp.int32(0))
        cand_nk[pl.ds(mc, 16)] = jnp.zeros((16,), jnp.int32)
        cand_pay[pl.ds(mc, 16)] = jnp.zeros((16,), jnp.int32)
        nvec = (mc + 15) >> 4

        _counting_pass(cand_nk, cand_pay, nk_a, pay_a, hist, 0, badj, nvec)
        _counting_pass(nk_a, pay_a, cand_nk, cand_pay, hist, LO_BITS, badj,
                       nvec)
        _counting_pass(cand_nk, cand_pay, nk_a, pay_a, hist, 2 * LO_BITS,
                       badj, nvec)

        def out_body(i, nv):
            v = nk_a[pl.ds(i * 16, 16)]
            p = pay_a[pl.ds(i * 16, 16)]
            valid = v > 0
            sco_v[pl.ds(i * 16, 16)] = jnp.where(
                valid, plsc.bitcast(v + KEY_BASE, jnp.float32),
                jnp.float32(-1.0))
            lbl_v[pl.ds(i * 16, 16)] = jnp.where(valid, p & 127, -1)
            row = jnp.where(valid, p >> 7, 0)
            k6 = (i * 16 + lax.iota(jnp.int32, 16)) * 6
            r6 = row * 6
            for cc in range(6):
                plsc.store_scatter(gidx, [k6 + cc], r6 + cc)
            return nv + jnp.max(plsc.all_reduce_population_count(valid))

        nvalid = lax.fori_loop(0, K // 16, out_body, jnp.int32(0))

        pltpu.sync_copy(sco_v, scores_out.at[b])
        pltpu.sync_copy(lbl_v, labels_out.at[b])

        pltpu.async_copy(boxes_hbm.at[b].at[gidx], box_v, sem).wait()

        s6 = nvalid * 6

        def fix_body(i, _):
            base = i * 16
            msk = (base + lax.iota(jnp.int32, 16)) >= s6
            v = box_v[pl.ds(base, 16)]
            box_v[pl.ds(base, 16)] = jnp.where(msk, jnp.float32(-1.0), v)
            return 0

        lax.fori_loop(s6 >> 4, (K * 6) // 16, fix_body, 0)

        pltpu.sync_copy(box_v, boxes_out.at[b])


def _sc_stage(nk, pay, boxes_pad):
    mesh = plsc.VectorSubcoreMesh(core_axis_name="c", subcore_axis_name="s")
    kern = pl.kernel(
        _sc_body,
        out_type=(
            jax.ShapeDtypeStruct((B, K * 6), jnp.float32),
            jax.ShapeDtypeStruct((B, K), jnp.float32),
            jax.ShapeDtypeStruct((B, K), jnp.int32),
        ),
        mesh=mesh,
        compiler_params=pltpu.CompilerParams(
            needs_layout_passes=False, use_tc_tiling_on_sc=False),
        scratch_types=[
            pltpu.VMEM((N2,), jnp.int32),
            pltpu.VMEM((N2,), jnp.int32),
            pltpu.VMEM((N2 + 16,), jnp.int32),
            pltpu.VMEM((N2 + 16,), jnp.int32),
            pltpu.VMEM((NUM_BINS,), jnp.int32),
            pltpu.VMEM((K * 6,), jnp.int32),
            pltpu.VMEM((K,), jnp.float32),
            pltpu.VMEM((K,), jnp.int32),
            pltpu.VMEM((K * 6,), jnp.float32),
            pltpu.SemaphoreType.DMA,
        ],
    )
    return kern(nk, pay, boxes_pad)


def kernel(boxes, classification):
    nk3, pay3 = _tc_stage(classification)
    boxes_flat_in = boxes.reshape(B, N * 6)
    boxes_flat, scores_out, labels_out = _sc_stage(nk3, pay3, boxes_flat_in)
    return boxes_flat.reshape(B, K, 6), scores_out, labels_out


# speedup vs baseline: 1.9054x; 1.1478x over previous
import functools

import jax
import jax.numpy as jnp
from jax import lax
from jax.experimental import pallas as pl
from jax.experimental.pallas import tpu as pltpu, tpu_sc as plsc

B = 8
N = 20000
C = 80
K = 2000
PAD_ROWS = 8
NP_ = N + PAD_ROWS

SCORE_THRESHOLD = 0.05
KEY_BASE = 0x3D4CCCCD
DIGIT_BITS = 13
DIGIT_MASK = (1 << DIGIT_BITS) - 1
NUM_BINS = 1 << DIGIT_BITS

BN = 2000
CH = 2048
NCHUNK = N // BN
N2 = NCHUNK * CH


def _tc_body(cls_ref, box_ref, nk_ref, pay_ref, bt_ref):
    j = pl.program_id(0)
    bi = j * CH + lax.broadcasted_iota(jnp.int32, (1, BN), 1)
    rows_nk = []
    rows_pay = []
    for bb in range(B):
        x = cls_ref[bb]
        xt = x.T
        m = jnp.max(xt, axis=0, keepdims=True)
        key = lax.bitcast_convert_type(m, jnp.int32)
        valid = m > jnp.float32(SCORE_THRESHOLD)
        nk = jnp.where(valid, key - KEY_BASE, 0)
        ir = lax.broadcasted_iota(jnp.int32, (C, BN), 0)
        lbl = jnp.min(
            jnp.where(xt == m, ir, jnp.int32(C * 2)), axis=0, keepdims=True
        )
        rows_nk.append(nk)
        rows_pay.append((bi << 7) | lbl)
        bt = box_ref[bb].T
        bt8 = jnp.concatenate(
            [bt, jnp.zeros((8 - 6, BN), jnp.float32)], 0)
        bt_ref[0, bb] = jnp.concatenate(
            [bt8, jnp.zeros((8, CH - BN), jnp.float32)], 1)
    zpad = jnp.zeros((B, CH - BN), jnp.int32)
    nk_ref[0] = jnp.concatenate([jnp.concatenate(rows_nk, 0), zpad], 1)
    pay_ref[0] = jnp.concatenate([jnp.concatenate(rows_pay, 0), zpad], 1)


def _tc_stage(classification, boxes):
    grid = (N // BN,)
    return pl.pallas_call(
        _tc_body,
        grid=grid,
        in_specs=[
            pl.BlockSpec((B, BN, C), lambda j: (0, j, 0)),
            pl.BlockSpec((B, BN, 6), lambda j: (0, j, 0)),
        ],
        out_specs=[
            pl.BlockSpec((1, B, CH), lambda j: (j, 0, 0)),
            pl.BlockSpec((1, B, CH), lambda j: (j, 0, 0)),
            pl.BlockSpec((1, B, 8, CH), lambda j: (j, 0, 0, 0)),
        ],
        out_shape=[
            jax.ShapeDtypeStruct((N // BN, B, CH), jnp.int32),
            jax.ShapeDtypeStruct((N // BN, B, CH), jnp.int32),
            jax.ShapeDtypeStruct((N // BN, B, 8, CH), jnp.float32),
        ],
    )(classification, boxes)


LO_BITS = 9
LO_MASK = (1 << LO_BITS) - 1
LO_BINS = 1 << LO_BITS


def _counting_pass(src_nk, src_pay, dst_nk, dst_pay, hist, shift, badj, nvec):
    def zero_body(i, _):
        hist[pl.ds(i * 16, 16)] = jnp.zeros((16,), jnp.int32)
        return 0

    lax.fori_loop(0, LO_BINS // 16, zero_body, 0)

    def hist_body(i, _):
        v = src_nk[pl.ds(i * 16, 16)]
        d = LO_MASK - ((v >> shift) & LO_MASK)
        cnt, last = plsc.scan_count(d)
        total = cnt - badj + 1
        plsc.addupdate_scatter(hist, [d], total, mask=last)
        return 0

    lax.fori_loop(0, nvec, hist_body, 0)

    def scan_body(i, carry):
        v = hist[pl.ds(i * 16, 16)]
        cs = plsc.cumsum(v)
        hist[pl.ds(i * 16, 16)] = cs - v + carry
        return carry + jnp.max(cs)

    lax.fori_loop(0, LO_BINS // 16, scan_body, jnp.int32(0))

    def perm_body(i, _):
        v = src_nk[pl.ds(i * 16, 16)]
        p = src_pay[pl.ds(i * 16, 16)]
        d = LO_MASK - ((v >> shift) & LO_MASK)
        cnt, last = plsc.scan_count(d)
        rank = cnt - badj
        cur = plsc.load_gather(hist, [d])
        off = cur + rank
        plsc.store_scatter(dst_nk, [off], v)
        plsc.store_scatter(dst_pay, [off], p)
        plsc.store_scatter(hist, [d], off + 1, mask=last)
        return 0

    lax.fori_loop(0, nvec, perm_body, 0)


def _sc_body(
    nk_hbm, pay_hbm, boxes_hbm,
    boxes_out, scores_out, labels_out,
    nk_a, pay_a, cand_nk, cand_pay, hist, gidx, sco_v, lbl_v, box_v, sem,
):
    c = lax.axis_index("c")
    s = lax.axis_index("s")
    wid = s * 2 + c

    @pl.when(wid < B)
    def _():
        b = wid
        cps = []
        for j in range(NCHUNK):
            cps.append(pltpu.async_copy(
                nk_hbm.at[j, b], nk_a.at[pl.ds(j * CH, CH)], sem))
            cps.append(pltpu.async_copy(
                pay_hbm.at[j, b], pay_a.at[pl.ds(j * CH, CH)], sem))
        cnt0, _ = plsc.scan_count(jnp.zeros((16,), jnp.int32))
        badj = jnp.min(cnt0)

        def zero_body(i, _):
            hist[pl.ds(i * 16, 16)] = jnp.zeros((16,), jnp.int32)
            return 0

        lax.fori_loop(0, NUM_BINS // 16, zero_body, 0)
        for cp in cps:
            cp.wait()

        def hist_body(i, _):
            for u in range(2):
                v = nk_a[pl.ds(i * 32 + u * 16, 16)]
                d = DIGIT_MASK - ((v >> DIGIT_BITS) & DIGIT_MASK)
                cnt, last = plsc.scan_count(d)
                plsc.addupdate_scatter(hist, [d], cnt - badj + 1, mask=last)
            return 0

        lax.fori_loop(0, N2 // 32, hist_body, 0)

        def cut_body(i, carry):
            cut, run = carry
            v = hist[pl.ds(i * 16, 16)]
            cs = plsc.cumsum(v) + run
            pc = jnp.max(plsc.all_reduce_population_count(cs >= K))
            cutl = i * 16 + (16 - pc)
            cut = jnp.minimum(
                cut, jnp.where(pc > 0, cutl, jnp.int32(NUM_BINS)))
            return cut, jnp.max(cs)

        cut, _ = lax.fori_loop(
            0, NUM_BINS // 16, cut_body,
            (jnp.int32(NUM_BINS), jnp.int32(0)))

        def compact_body(i, off):
            for u in range(2):
                v = nk_a[pl.ds(i * 32 + u * 16, 16)]
                p = pay_a[pl.ds(i * 32 + u * 16, 16)]
                d = DIGIT_MASK - ((v >> DIGIT_BITS) & DIGIT_MASK)
                m = d <= cut
                plsc.store_compressed(cand_nk.at[pl.ds(off, 16)], v, mask=m)
                plsc.store_compressed(cand_pay.at[pl.ds(off, 16)], p, mask=m)
                off = off + jnp.max(plsc.all_reduce_population_count(m))
            return off

        mc = lax.fori_loop(0, N2 // 32, compact_body, jnp.int32(0))
        cand_nk[pl.ds(mc, 16)] = jnp.zeros((16,), jnp.int32)
        cand_pay[pl.ds(mc, 16)] = jnp.zeros((16,), jnp.int32)
        nvec = (mc + 15) >> 4

        _counting_pass(cand_nk, cand_pay, nk_a, pay_a, hist, 0, badj, nvec)
        _counting_pass(nk_a, pay_a, cand_nk, cand_pay, hist, LO_BITS, badj,
                       nvec)
        _counting_pass(cand_nk, cand_pay, nk_a, pay_a, hist, 2 * LO_BITS,
                       badj, nvec)

        def out_body(i, nv):
            v = nk_a[pl.ds(i * 16, 16)]
            p = pay_a[pl.ds(i * 16, 16)]
            valid = v > 0
            sco_v[pl.ds(i * 16, 16)] = jnp.where(
                valid, plsc.bitcast(v + KEY_BASE, jnp.float32),
                jnp.float32(-1.0))
            lbl_v[pl.ds(i * 16, 16)] = jnp.where(valid, p & 127, -1)
            g2 = jnp.where(valid, p >> 7, 0)
            base = ((g2 >> 11) << 17) + (b << 14) + (g2 & 2047)
            k6 = (i * 16 + lax.iota(jnp.int32, 16)) * 6
            for cc in range(6):
                plsc.store_scatter(gidx, [k6 + cc], base + (cc << 11))
            return nv + jnp.max(plsc.all_reduce_population_count(valid))

        nvalid = lax.fori_loop(0, K // 16, out_body, jnp.int32(0))

        pltpu.sync_copy(sco_v, scores_out.at[b])
        pltpu.sync_copy(lbl_v, labels_out.at[b])

        pltpu.async_copy(boxes_hbm.at[gidx], box_v, sem).wait()

        s6 = nvalid * 6

        def fix_body(i, _):
            base = i * 16
            msk = (base + lax.iota(jnp.int32, 16)) >= s6
            v = box_v[pl.ds(base, 16)]
            box_v[pl.ds(base, 16)] = jnp.where(msk, jnp.float32(-1.0), v)
            return 0

        lax.fori_loop(s6 >> 4, (K * 6) // 16, fix_body, 0)

        pltpu.sync_copy(box_v, boxes_out.at[b])


def _sc_stage(nk, pay, boxes_pad):
    mesh = plsc.VectorSubcoreMesh(core_axis_name="c", subcore_axis_name="s")
    kern = pl.kernel(
        _sc_body,
        out_type=(
            jax.ShapeDtypeStruct((B, K * 6), jnp.float32),
            jax.ShapeDtypeStruct((B, K), jnp.float32),
            jax.ShapeDtypeStruct((B, K), jnp.int32),
        ),
        mesh=mesh,
        compiler_params=pltpu.CompilerParams(
            needs_layout_passes=False, use_tc_tiling_on_sc=False),
        scratch_types=[
            pltpu.VMEM((N2,), jnp.int32),
            pltpu.VMEM((N2,), jnp.int32),
            pltpu.VMEM((N2 + 16,), jnp.int32),
            pltpu.VMEM((N2 + 16,), jnp.int32),
            pltpu.VMEM((NUM_BINS,), jnp.int32),
            pltpu.VMEM((K * 6,), jnp.int32),
            pltpu.VMEM((K,), jnp.float32),
            pltpu.VMEM((K,), jnp.int32),
            pltpu.VMEM((K * 6,), jnp.float32),
            pltpu.SemaphoreType.DMA,
        ],
    )
    return kern(nk, pay, boxes_pad)


def kernel(boxes, classification):
    nk3, pay3, bt = _tc_stage(classification, boxes)
    boxes_flat, scores_out, labels_out = _sc_stage(
        nk3, pay3, bt.reshape(NCHUNK * B * 8 * CH))
    return boxes_flat.reshape(B, K, 6), scores_out, labels_out
